# baseline (speedup 1.0000x reference)
import jax
import jax.numpy as jnp
from jax.experimental import pallas as pl
from jax.experimental.pallas import tpu as pltpu

_T, _B, _N, _I = 12, 16, 512, 2
_H = 64
_L = 2
_S = 2
_K = 3
_NUM_MAT = 1 + _S * (_K - 1)


def _cheb_proj(x, sups, w, b2d):
    c = x.shape[1]
    acc = jnp.dot(x, w[0:c], preferred_element_type=jnp.float32)
    k = 1
    for sm in sups:
        t1 = jnp.dot(sm, x, preferred_element_type=jnp.float32)
        acc = acc + jnp.dot(t1, w[k * c:(k + 1) * c],
                            preferred_element_type=jnp.float32)
        k += 1
        t2 = 2.0 * jnp.dot(sm, t1, preferred_element_type=jnp.float32) - x
        acc = acc + jnp.dot(t2, w[k * c:(k + 1) * c],
                            preferred_element_type=jnp.float32)
        k += 1
    return acc + b2d


def _cell(inp, st, sups, w_ru, b_ru, w_h, b_h):
    x = jnp.concatenate([inp, st], axis=1)
    g = jax.nn.sigmoid(_cheb_proj(x, sups, w_ru, b_ru))
    r = g[:, :_H]
    u = g[:, _H:]
    x2 = jnp.concatenate([inp, r * st], axis=1)
    cand = jnp.tanh(_cheb_proj(x2, sups, w_h, b_h))
    return u * st + (1.0 - u) * cand


def _body(x_ref, sup_ref, wru0_ref, bru0_ref, wh0_ref, bh0_ref,
          wru1_ref, bru1_ref, wh1_ref, bh1_ref, out_ref):
    sups = [sup_ref[0], sup_ref[1]]
    wru0 = wru0_ref[:, :]
    bru0 = bru0_ref[:, :]
    wh0 = wh0_ref[:, :]
    bh0 = bh0_ref[:, :]
    wru1 = wru1_ref[:, :]
    bru1 = bru1_ref[:, :]
    wh1 = wh1_ref[:, :]
    bh1 = bh1_ref[:, :]

    def step(t, carry):
        s0, s1 = carry
        inp = x_ref[t, 0]
        o0 = _cell(inp, s0, sups, wru0, bru0, wh0, bh0)
        o1 = _cell(o0, s1, sups, wru1, bru1, wh1, bh1)
        return (o0, o1)

    z = jnp.zeros((_N, _H), jnp.float32)
    s0, s1 = jax.lax.fori_loop(0, _T, step, (z, z))
    out_ref[0, 0] = s0
    out_ref[1, 0] = s1


def kernel(inputs, supports, W_ru_0, b_ru_0, W_h_0, b_h_0,
           W_ru_1, b_ru_1, W_h_1, b_h_1):
    in0 = (_I + _H) * _NUM_MAT
    in1 = (_H + _H) * _NUM_MAT
    out = pl.pallas_call(
        _body,
        grid=(_B,),
        in_specs=[
            pl.BlockSpec((_T, 1, _N, _I), lambda b: (0, b, 0, 0)),
            pl.BlockSpec((_S, _N, _N), lambda b: (0, 0, 0)),
            pl.BlockSpec((in0, 2 * _H), lambda b: (0, 0)),
            pl.BlockSpec((1, 2 * _H), lambda b: (0, 0)),
            pl.BlockSpec((in0, _H), lambda b: (0, 0)),
            pl.BlockSpec((1, _H), lambda b: (0, 0)),
            pl.BlockSpec((in1, 2 * _H), lambda b: (0, 0)),
            pl.BlockSpec((1, 2 * _H), lambda b: (0, 0)),
            pl.BlockSpec((in1, _H), lambda b: (0, 0)),
            pl.BlockSpec((1, _H), lambda b: (0, 0)),
        ],
        out_specs=pl.BlockSpec((_L, 1, _N, _H), lambda b: (0, b, 0, 0)),
        out_shape=jax.ShapeDtypeStruct((_L, _B, _N, _H), jnp.float32),
        compiler_params=pltpu.CompilerParams(
            dimension_semantics=("parallel",)),
    )(inputs, supports,
      W_ru_0, b_ru_0.reshape(1, -1), W_h_0, b_h_0.reshape(1, -1),
      W_ru_1, b_ru_1.reshape(1, -1), W_h_1, b_h_1.reshape(1, -1))
    return out


# speedup vs baseline: 1.6904x; 1.6904x over previous
import jax
import jax.numpy as jnp
from jax.experimental import pallas as pl
from jax.experimental.pallas import tpu as pltpu

_T, _B, _N, _I = 12, 16, 512, 2
_H = 64
_L = 2
_S = 2
_K = 3
_NUM_MAT = 1 + _S * (_K - 1)
_P = 2
_C0 = _I + _H
_C1 = _H + _H


def _diffuse(s1, s2, x):
    t1a = jnp.dot(s1, x, preferred_element_type=jnp.float32)
    t2a = 2.0 * jnp.dot(s1, t1a, preferred_element_type=jnp.float32) - x
    t1b = jnp.dot(s2, x, preferred_element_type=jnp.float32)
    t2b = 2.0 * jnp.dot(s2, t1b, preferred_element_type=jnp.float32) - x
    return [x, t1a, t2a, t1b, t2b]


def _proj(mats, col0, w, acc):
    for k, m in enumerate(mats):
        acc = acc + jnp.dot(m[:, col0:col0 + _H], w[k * _H:(k + 1) * _H],
                            preferred_element_type=jnp.float32)
    return acc


def _body(x_ref, sup_ref, w0i_ref, wru0_ref, wh0_ref,
          w1i_ref, wru1_ref, wh1_ref, b0_ref, b1_ref,
          out_ref, g0c_ref):
    s1m = sup_ref[0]
    s2m = sup_ref[1]
    w0i = w0i_ref[:, :]
    wru0 = wru0_ref[:, :]
    wh0 = wh0_ref[:, :]
    w1i = w1i_ref[:, :]
    wru1 = wru1_ref[:, :]
    wh1 = wh1_ref[:, :]
    b0c = b0_ref[:, :]
    b1c = b1_ref[:, :]

    ia = jnp.concatenate([x_ref[0], x_ref[1]], axis=1)
    imats = _diffuse(s1m, s2m, ia)
    ti = _T * _I
    for t in range(_T):
        for j in range(_P):
            c = j * ti + _I * t
            cols = jnp.concatenate([m[:, c:c + _I] for m in imats], axis=1)
            g0c_ref[t, j] = jnp.dot(cols, w0i,
                                    preferred_element_type=jnp.float32) + b0c

    def step(t, carry):
        s01, s11 = carry

        st = jnp.concatenate([s01, s11], axis=1)
        smats = _diffuse(s1m, s2m, st)

        gs = [jax.nn.sigmoid(_proj(smats, j * _H, wru0,
                                   g0c_ref[t, j][:, :2 * _H]))
              for j in range(_P)]
        r0 = jnp.concatenate([g[:, :_H] for g in gs], axis=1)
        u0 = jnp.concatenate([g[:, _H:] for g in gs], axis=1)

        cmats = _diffuse(s1m, s2m, r0 * s01)
        cs = [jnp.tanh(_proj(cmats, j * _H, wh0,
                             g0c_ref[t, j][:, 2 * _H:]))
              for j in range(_P)]
        o0 = u0 * s01 + (1.0 - u0) * jnp.concatenate(cs, axis=1)

        pmats = _diffuse(s1m, s2m, o0)
        gi = [_proj(pmats, j * _H, w1i, b1c) for j in range(_P)]

        g1 = [jax.nn.sigmoid(_proj(smats, _P * _H + j * _H, wru1,
                                   gi[j][:, :2 * _H]))
              for j in range(_P)]
        r1 = jnp.concatenate([g[:, :_H] for g in g1], axis=1)
        u1 = jnp.concatenate([g[:, _H:] for g in g1], axis=1)

        qmats = _diffuse(s1m, s2m, r1 * s11)
        c1 = [jnp.tanh(_proj(qmats, j * _H, wh1, gi[j][:, 2 * _H:]))
              for j in range(_P)]
        s1n = u1 * s11 + (1.0 - u1) * jnp.concatenate(c1, axis=1)
        return (o0, s1n)

    z = jnp.zeros((_N, _P * _H), jnp.float32)
    s01, s11 = jax.lax.fori_loop(0, _T, step, (z, z))
    for j in range(_P):
        out_ref[0, j] = s01[:, j * _H:(j + 1) * _H]
        out_ref[1, j] = s11[:, j * _H:(j + 1) * _H]


def kernel(inputs, supports, W_ru_0, b_ru_0, W_h_0, b_h_0,
           W_ru_1, b_ru_1, W_h_1, b_h_1):
    x_r = inputs.transpose(1, 2, 0, 3).reshape(_B, _N, _T * _I)
    w0i = jnp.concatenate(
        [jnp.concatenate([W_ru_0[k * _C0:k * _C0 + _I],
                          W_h_0[k * _C0:k * _C0 + _I]], axis=1)
         for k in range(_NUM_MAT)], axis=0)
    wru0 = jnp.concatenate(
        [W_ru_0[k * _C0 + _I:(k + 1) * _C0] for k in range(_NUM_MAT)],
        axis=0)
    wh0 = jnp.concatenate(
        [W_h_0[k * _C0 + _I:(k + 1) * _C0] for k in range(_NUM_MAT)],
        axis=0)
    w1i = jnp.concatenate(
        [jnp.concatenate([W_ru_1[k * _C1:k * _C1 + _H],
                          W_h_1[k * _C1:k * _C1 + _H]], axis=1)
         for k in range(_NUM_MAT)], axis=0)
    wru1 = jnp.concatenate(
        [W_ru_1[k * _C1 + _H:(k + 1) * _C1] for k in range(_NUM_MAT)],
        axis=0)
    wh1 = jnp.concatenate(
        [W_h_1[k * _C1 + _H:(k + 1) * _C1] for k in range(_NUM_MAT)],
        axis=0)
    b0c = jnp.concatenate([b_ru_0, b_h_0]).reshape(1, 3 * _H)
    b1c = jnp.concatenate([b_ru_1, b_h_1]).reshape(1, 3 * _H)

    out = pl.pallas_call(
        _body,
        grid=(_B // _P,),
        in_specs=[
            pl.BlockSpec((_P, _N, _T * _I), lambda p: (p, 0, 0)),
            pl.BlockSpec((_S, _N, _N), lambda p: (0, 0, 0)),
            pl.BlockSpec((_NUM_MAT * _I, 3 * _H), lambda p: (0, 0)),
            pl.BlockSpec((_NUM_MAT * _H, 2 * _H), lambda p: (0, 0)),
            pl.BlockSpec((_NUM_MAT * _H, _H), lambda p: (0, 0)),
            pl.BlockSpec((_NUM_MAT * _H, 3 * _H), lambda p: (0, 0)),
            pl.BlockSpec((_NUM_MAT * _H, 2 * _H), lambda p: (0, 0)),
            pl.BlockSpec((_NUM_MAT * _H, _H), lambda p: (0, 0)),
            pl.BlockSpec((1, 3 * _H), lambda p: (0, 0)),
            pl.BlockSpec((1, 3 * _H), lambda p: (0, 0)),
        ],
        out_specs=pl.BlockSpec((_L, _P, _N, _H), lambda p: (0, p, 0, 0)),
        out_shape=jax.ShapeDtypeStruct((_L, _B, _N, _H), jnp.float32),
        scratch_shapes=[pltpu.VMEM((_T, _P, _N, 3 * _H), jnp.float32)],
        compiler_params=pltpu.CompilerParams(
            dimension_semantics=("parallel",)),
    )(x_r, supports, w0i, wru0, wh0, w1i, wru1, wh1, b0c, b1c)
    return out


# speedup vs baseline: 1.6980x; 1.0045x over previous
import jax
import jax.numpy as jnp
from jax.experimental import pallas as pl
from jax.experimental.pallas import tpu as pltpu

_T, _B, _N, _I = 12, 16, 512, 2
_H = 64
_L = 2
_S = 2
_K = 3
_NUM_MAT = 1 + _S * (_K - 1)
_P = 2
_C0 = _I + _H
_C1 = _H + _H


def _diffuse(s1, s2, x):
    t1a = jnp.dot(s1, x, preferred_element_type=jnp.float32)
    t2a = 2.0 * jnp.dot(s1, t1a, preferred_element_type=jnp.float32) - x
    t1b = jnp.dot(s2, x, preferred_element_type=jnp.float32)
    t2b = 2.0 * jnp.dot(s2, t1b, preferred_element_type=jnp.float32) - x
    return [x, t1a, t2a, t1b, t2b]


def _proj(mats, w, acc):
    for k, m in enumerate(mats):
        acc = acc + jnp.dot(m, w[k * 2 * _H:(k + 1) * 2 * _H],
                            preferred_element_type=jnp.float32)
    return acc


def _body(x_ref, sup_ref, w0i_ref, wg0_ref, wc0_ref,
          w1i_ref, wg1_ref, wc1_ref, b0_ref, b1_ref,
          out_ref, g0c_ref):
    s1m = sup_ref[0]
    s2m = sup_ref[1]
    w0i = w0i_ref[:, :]
    wg0 = wg0_ref[:, :]
    wc0 = wc0_ref[:, :]
    w1i = w1i_ref[:, :]
    wg1 = wg1_ref[:, :]
    wc1 = wc1_ref[:, :]
    b0c = b0_ref[:, :]
    b1c = b1_ref[:, :]

    ia = jnp.concatenate([x_ref[0], x_ref[1]], axis=1)
    imats = _diffuse(s1m, s2m, ia)
    ti = _T * _I
    for t in range(_T):
        cols = jnp.concatenate(
            [m[:, j * ti + _I * t:j * ti + _I * t + _I]
             for j in range(_P) for m in imats], axis=1)
        g0c_ref[t] = jnp.dot(cols, w0i,
                             preferred_element_type=jnp.float32) + b0c

    def step(t, carry):
        s01, s11 = carry

        st = jnp.concatenate([s01, s11], axis=1)
        smats = _diffuse(s1m, s2m, st)
        s0mats = [m[:, :_P * _H] for m in smats]
        s1mats = [m[:, _P * _H:] for m in smats]

        g0 = jax.nn.sigmoid(_proj(s0mats, wg0, g0c_ref[t, :, :4 * _H]))
        r0 = g0[:, :_P * _H]
        u0 = g0[:, _P * _H:]

        cmats = _diffuse(s1m, s2m, r0 * s01)
        cand0 = jnp.tanh(_proj(cmats, wc0, g0c_ref[t, :, 4 * _H:]))
        o0 = u0 * s01 + (1.0 - u0) * cand0

        pmats = _diffuse(s1m, s2m, o0)
        gi = _proj(pmats, w1i, b1c)

        g1 = jax.nn.sigmoid(_proj(s1mats, wg1, gi[:, :4 * _H]))
        r1 = g1[:, :_P * _H]
        u1 = g1[:, _P * _H:]

        qmats = _diffuse(s1m, s2m, r1 * s11)
        cand1 = jnp.tanh(_proj(qmats, wc1, gi[:, 4 * _H:]))
        s1n = u1 * s11 + (1.0 - u1) * cand1
        return (o0, s1n)

    z = jnp.zeros((_N, _P * _H), jnp.float32)
    s01, s11 = jax.lax.fori_loop(0, _T, step, (z, z))
    for j in range(_P):
        out_ref[0, j] = s01[:, j * _H:(j + 1) * _H]
        out_ref[1, j] = s11[:, j * _H:(j + 1) * _H]


def _bd_gate(w):
    r, u = w[:, :_H], w[:, _H:]
    z = jnp.zeros_like(r)
    return jnp.concatenate(
        [jnp.concatenate([r, z, u, z], axis=1),
         jnp.concatenate([z, r, z, u], axis=1)], axis=0)


def _bd_cand(w):
    z = jnp.zeros_like(w)
    return jnp.concatenate(
        [jnp.concatenate([w, z], axis=1),
         jnp.concatenate([z, w], axis=1)], axis=0)


def _bd_fused(wr, wh):
    r, u = wr[:, :_H], wr[:, _H:]
    z = jnp.zeros_like(r)
    return jnp.concatenate(
        [jnp.concatenate([r, z, u, z, wh, z], axis=1),
         jnp.concatenate([z, r, z, u, z, wh], axis=1)], axis=0)


def kernel(inputs, supports, W_ru_0, b_ru_0, W_h_0, b_h_0,
           W_ru_1, b_ru_1, W_h_1, b_h_1):
    x_r = inputs.transpose(1, 2, 0, 3).reshape(_B, _N, _T * _I)

    wg0 = jnp.concatenate(
        [_bd_gate(W_ru_0[k * _C0 + _I:(k + 1) * _C0])
         for k in range(_NUM_MAT)], axis=0)
    wc0 = jnp.concatenate(
        [_bd_cand(W_h_0[k * _C0 + _I:(k + 1) * _C0])
         for k in range(_NUM_MAT)], axis=0)
    wg1 = jnp.concatenate(
        [_bd_gate(W_ru_1[k * _C1 + _H:(k + 1) * _C1])
         for k in range(_NUM_MAT)], axis=0)
    wc1 = jnp.concatenate(
        [_bd_cand(W_h_1[k * _C1 + _H:(k + 1) * _C1])
         for k in range(_NUM_MAT)], axis=0)
    w1i = jnp.concatenate(
        [_bd_fused(W_ru_1[k * _C1:k * _C1 + _H],
                   W_h_1[k * _C1:k * _C1 + _H])
         for k in range(_NUM_MAT)], axis=0)

    ri = jnp.concatenate(
        [W_ru_0[k * _C0:k * _C0 + _I, :_H] for k in range(_NUM_MAT)], axis=0)
    ui = jnp.concatenate(
        [W_ru_0[k * _C0:k * _C0 + _I, _H:] for k in range(_NUM_MAT)], axis=0)
    ci = jnp.concatenate(
        [W_h_0[k * _C0:k * _C0 + _I] for k in range(_NUM_MAT)], axis=0)
    zi = jnp.zeros_like(ri)
    zc = jnp.zeros_like(ci)
    w0i = jnp.concatenate(
        [jnp.concatenate([ri, zi, ui, zi, ci, zc], axis=1),
         jnp.concatenate([zi, ri, zi, ui, zc, ci], axis=1)],
        axis=0)

    b0c = jnp.concatenate([b_ru_0[:_H], b_ru_0[:_H], b_ru_0[_H:],
                           b_ru_0[_H:], b_h_0, b_h_0]).reshape(1, 6 * _H)
    b1c = jnp.concatenate([b_ru_1[:_H], b_ru_1[:_H], b_ru_1[_H:],
                           b_ru_1[_H:], b_h_1, b_h_1]).reshape(1, 6 * _H)

    out = pl.pallas_call(
        _body,
        grid=(_B // _P,),
        in_specs=[
            pl.BlockSpec((_P, _N, _T * _I), lambda p: (p, 0, 0)),
            pl.BlockSpec((_S, _N, _N), lambda p: (0, 0, 0)),
            pl.BlockSpec((_P * _NUM_MAT * _I, 6 * _H), lambda p: (0, 0)),
            pl.BlockSpec((_NUM_MAT * 2 * _H, 4 * _H), lambda p: (0, 0)),
            pl.BlockSpec((_NUM_MAT * 2 * _H, 2 * _H), lambda p: (0, 0)),
            pl.BlockSpec((_NUM_MAT * 2 * _H, 6 * _H), lambda p: (0, 0)),
            pl.BlockSpec((_NUM_MAT * 2 * _H, 4 * _H), lambda p: (0, 0)),
            pl.BlockSpec((_NUM_MAT * 2 * _H, 2 * _H), lambda p: (0, 0)),
            pl.BlockSpec((1, 6 * _H), lambda p: (0, 0)),
            pl.BlockSpec((1, 6 * _H), lambda p: (0, 0)),
        ],
        out_specs=pl.BlockSpec((_L, _P, _N, _H), lambda p: (0, p, 0, 0)),
        out_shape=jax.ShapeDtypeStruct((_L, _B, _N, _H), jnp.float32),
        scratch_shapes=[pltpu.VMEM((_T, _N, 6 * _H), jnp.float32)],
        compiler_params=pltpu.CompilerParams(
            dimension_semantics=("parallel",)),
    )(x_r, supports, w0i, wg0, wc0, w1i, wg1, wc1, b0c, b1c)
    return out


# speedup vs baseline: 1.7576x; 1.0351x over previous
import jax
import jax.numpy as jnp
from jax.experimental import pallas as pl
from jax.experimental.pallas import tpu as pltpu

_T, _B, _N, _I = 12, 16, 512, 2
_H = 64
_L = 2
_S = 2
_K = 3
_NUM_MAT = 1 + _S * (_K - 1)
_P = 2
_C0 = _I + _H
_C1 = _H + _H


def _diffuse(s1, s2, x):
    xb = x.astype(jnp.bfloat16)
    t1a = jnp.dot(s1, xb, preferred_element_type=jnp.float32)
    t1ab = t1a.astype(jnp.bfloat16)
    t2ab = (2.0 * jnp.dot(s1, t1ab, preferred_element_type=jnp.float32)
            - x).astype(jnp.bfloat16)
    t1b = jnp.dot(s2, xb, preferred_element_type=jnp.float32)
    t1bb = t1b.astype(jnp.bfloat16)
    t2bb = (2.0 * jnp.dot(s2, t1bb, preferred_element_type=jnp.float32)
            - x).astype(jnp.bfloat16)
    return [xb, t1ab, t2ab, t1bb, t2bb]


def _proj(mats, w, acc):
    for k, m in enumerate(mats):
        acc = acc + jnp.dot(m, w[k * 2 * _H:(k + 1) * 2 * _H],
                            preferred_element_type=jnp.float32)
    return acc


def _body(x_ref, sup_ref, w0i_ref, wg0_ref, wc0_ref,
          w1i_ref, wg1_ref, wc1_ref, b0_ref, b1_ref,
          out_ref, g0c_ref):
    s1m = sup_ref[0]
    s2m = sup_ref[1]
    w0i = w0i_ref[:, :]
    wg0 = wg0_ref[:, :]
    wc0 = wc0_ref[:, :]
    w1i = w1i_ref[:, :]
    wg1 = wg1_ref[:, :]
    wc1 = wc1_ref[:, :]
    b0c = b0_ref[:, :]
    b1c = b1_ref[:, :]

    ia = jnp.concatenate([x_ref[0], x_ref[1]], axis=1)
    imats = _diffuse(s1m, s2m, ia)
    ti = _T * _I
    for t in range(_T):
        cols = jnp.concatenate(
            [m[:, j * ti + _I * t:j * ti + _I * t + _I]
             for j in range(_P) for m in imats], axis=1)
        g0c_ref[t] = jnp.dot(cols, w0i,
                             preferred_element_type=jnp.float32) + b0c

    def step(t, carry):
        s01, s11 = carry

        st = jnp.concatenate([s01, s11], axis=1)
        smats = _diffuse(s1m, s2m, st)
        s0mats = [m[:, :_P * _H] for m in smats]
        s1mats = [m[:, _P * _H:] for m in smats]

        g0 = jax.nn.sigmoid(_proj(s0mats, wg0, g0c_ref[t, :, :4 * _H]))
        r0 = g0[:, :_P * _H]
        u0 = g0[:, _P * _H:]

        cmats = _diffuse(s1m, s2m, r0 * s01)
        cand0 = jnp.tanh(_proj(cmats, wc0, g0c_ref[t, :, 4 * _H:]))
        o0 = u0 * s01 + (1.0 - u0) * cand0

        pmats = _diffuse(s1m, s2m, o0)
        gi = _proj(pmats, w1i, b1c)

        g1 = jax.nn.sigmoid(_proj(s1mats, wg1, gi[:, :4 * _H]))
        r1 = g1[:, :_P * _H]
        u1 = g1[:, _P * _H:]

        qmats = _diffuse(s1m, s2m, r1 * s11)
        cand1 = jnp.tanh(_proj(qmats, wc1, gi[:, 4 * _H:]))
        s1n = u1 * s11 + (1.0 - u1) * cand1
        return (o0, s1n)

    z = jnp.zeros((_N, _P * _H), jnp.float32)
    s01, s11 = jax.lax.fori_loop(0, _T, step, (z, z))
    for j in range(_P):
        out_ref[0, j] = s01[:, j * _H:(j + 1) * _H]
        out_ref[1, j] = s11[:, j * _H:(j + 1) * _H]


def _bd_gate(w):
    r, u = w[:, :_H], w[:, _H:]
    z = jnp.zeros_like(r)
    return jnp.concatenate(
        [jnp.concatenate([r, z, u, z], axis=1),
         jnp.concatenate([z, r, z, u], axis=1)], axis=0)


def _bd_cand(w):
    z = jnp.zeros_like(w)
    return jnp.concatenate(
        [jnp.concatenate([w, z], axis=1),
         jnp.concatenate([z, w], axis=1)], axis=0)


def _bd_fused(wr, wh):
    r, u = wr[:, :_H], wr[:, _H:]
    z = jnp.zeros_like(r)
    return jnp.concatenate(
        [jnp.concatenate([r, z, u, z, wh, z], axis=1),
         jnp.concatenate([z, r, z, u, z, wh], axis=1)], axis=0)


def kernel(inputs, supports, W_ru_0, b_ru_0, W_h_0, b_h_0,
           W_ru_1, b_ru_1, W_h_1, b_h_1):
    x_r = inputs.transpose(1, 2, 0, 3).reshape(_B, _N, _T * _I)

    wg0 = jnp.concatenate(
        [_bd_gate(W_ru_0[k * _C0 + _I:(k + 1) * _C0])
         for k in range(_NUM_MAT)], axis=0)
    wc0 = jnp.concatenate(
        [_bd_cand(W_h_0[k * _C0 + _I:(k + 1) * _C0])
         for k in range(_NUM_MAT)], axis=0)
    wg1 = jnp.concatenate(
        [_bd_gate(W_ru_1[k * _C1 + _H:(k + 1) * _C1])
         for k in range(_NUM_MAT)], axis=0)
    wc1 = jnp.concatenate(
        [_bd_cand(W_h_1[k * _C1 + _H:(k + 1) * _C1])
         for k in range(_NUM_MAT)], axis=0)
    w1i = jnp.concatenate(
        [_bd_fused(W_ru_1[k * _C1:k * _C1 + _H],
                   W_h_1[k * _C1:k * _C1 + _H])
         for k in range(_NUM_MAT)], axis=0)

    ri = jnp.concatenate(
        [W_ru_0[k * _C0:k * _C0 + _I, :_H] for k in range(_NUM_MAT)], axis=0)
    ui = jnp.concatenate(
        [W_ru_0[k * _C0:k * _C0 + _I, _H:] for k in range(_NUM_MAT)], axis=0)
    ci = jnp.concatenate(
        [W_h_0[k * _C0:k * _C0 + _I] for k in range(_NUM_MAT)], axis=0)
    zi = jnp.zeros_like(ri)
    zc = jnp.zeros_like(ci)
    w0i = jnp.concatenate(
        [jnp.concatenate([ri, zi, ui, zi, ci, zc], axis=1),
         jnp.concatenate([zi, ri, zi, ui, zc, ci], axis=1)],
        axis=0)

    b0c = jnp.concatenate([b_ru_0[:_H], b_ru_0[:_H], b_ru_0[_H:],
                           b_ru_0[_H:], b_h_0, b_h_0]).reshape(1, 6 * _H)
    b1c = jnp.concatenate([b_ru_1[:_H], b_ru_1[:_H], b_ru_1[_H:],
                           b_ru_1[_H:], b_h_1, b_h_1]).reshape(1, 6 * _H)

    out = pl.pallas_call(
        _body,
        grid=(_B // _P,),
        in_specs=[
            pl.BlockSpec((_P, _N, _T * _I), lambda p: (p, 0, 0)),
            pl.BlockSpec((_S, _N, _N), lambda p: (0, 0, 0)),
            pl.BlockSpec((_P * _NUM_MAT * _I, 6 * _H), lambda p: (0, 0)),
            pl.BlockSpec((_NUM_MAT * 2 * _H, 4 * _H), lambda p: (0, 0)),
            pl.BlockSpec((_NUM_MAT * 2 * _H, 2 * _H), lambda p: (0, 0)),
            pl.BlockSpec((_NUM_MAT * 2 * _H, 6 * _H), lambda p: (0, 0)),
            pl.BlockSpec((_NUM_MAT * 2 * _H, 4 * _H), lambda p: (0, 0)),
            pl.BlockSpec((_NUM_MAT * 2 * _H, 2 * _H), lambda p: (0, 0)),
            pl.BlockSpec((1, 6 * _H), lambda p: (0, 0)),
            pl.BlockSpec((1, 6 * _H), lambda p: (0, 0)),
        ],
        out_specs=pl.BlockSpec((_L, _P, _N, _H), lambda p: (0, p, 0, 0)),
        out_shape=jax.ShapeDtypeStruct((_L, _B, _N, _H), jnp.float32),
        scratch_shapes=[pltpu.VMEM((_T, _N, 6 * _H), jnp.float32)],
        compiler_params=pltpu.CompilerParams(
            dimension_semantics=("parallel",)),
    )(x_r, supports.astype(jnp.bfloat16),
      w0i.astype(jnp.bfloat16), wg0.astype(jnp.bfloat16),
      wc0.astype(jnp.bfloat16), w1i.astype(jnp.bfloat16),
      wg1.astype(jnp.bfloat16), wc1.astype(jnp.bfloat16), b0c, b1c)
    return out


# speedup vs baseline: 2.2981x; 1.3075x over previous
import jax
import jax.numpy as jnp
from jax.experimental import pallas as pl
from jax.experimental.pallas import tpu as pltpu

_T, _B, _N, _I = 12, 16, 512, 2
_H = 64
_L = 2
_S = 2
_K = 3
_NUM_MAT = 1 + _S * (_K - 1)
_P = 2
_C0 = _I + _H
_C1 = _H + _H
_PH = _P * _H


def _diffuse(s1, s2, x):
    xb = x.astype(jnp.bfloat16)
    t1a = jnp.dot(s1, xb, preferred_element_type=jnp.float32)
    t1ab = t1a.astype(jnp.bfloat16)
    t2ab = (2.0 * jnp.dot(s1, t1ab, preferred_element_type=jnp.float32)
            - x).astype(jnp.bfloat16)
    t1b = jnp.dot(s2, xb, preferred_element_type=jnp.float32)
    t1bb = t1b.astype(jnp.bfloat16)
    t2bb = (2.0 * jnp.dot(s2, t1bb, preferred_element_type=jnp.float32)
            - x).astype(jnp.bfloat16)
    return [xb, t1ab, t2ab, t1bb, t2bb]


def _proj(mats, w, acc):
    for k, m in enumerate(mats):
        acc = acc + jnp.dot(m, w[k * _PH:(k + 1) * _PH],
                            preferred_element_type=jnp.float32)
    return acc


def _body(x_ref, sup_ref, w0i_ref, wg0_ref, wc0_ref,
          w1i_ref, wg1_ref, wc1_ref, b0_ref, b1_ref,
          out_ref, g0c_ref):
    s1m = sup_ref[0]
    s2m = sup_ref[1]
    w0i = w0i_ref[:, :]
    wg0 = wg0_ref[:, :]
    wc0 = wc0_ref[:, :]
    w1i = w1i_ref[:, :]
    wg1 = wg1_ref[:, :]
    wc1 = wc1_ref[:, :]
    b0c = b0_ref[:, :]
    b1c = b1_ref[:, :]

    imats = _diffuse(s1m, s2m, x_ref[0])
    for t in range(_T):
        cols = jnp.concatenate(
            [m[:, _P * _I * t:_P * _I * (t + 1)] for m in imats], axis=1)
        g0c_ref[t] = jnp.dot(cols, w0i,
                             preferred_element_type=jnp.float32) + b0c
    g0c_ref[_T] = jnp.zeros((_N, 6 * _H), jnp.float32)

    g00 = g0c_ref[0]
    u00 = jax.nn.sigmoid(g00[:, _PH:2 * _PH])
    o00 = (1.0 - u00) * jnp.tanh(g00[:, 4 * _H:])

    def step(t, carry):
        os, _ = carry
        o0f = os[:, :_PH]
        s1f = os[:, _PH:]

        dmats = _diffuse(s1m, s2m, os)
        dm0 = [m[:, :_PH] for m in dmats]
        dm1 = [m[:, _PH:] for m in dmats]

        gi = _proj(dm0, w1i, b1c)
        g1 = jax.nn.sigmoid(_proj(dm1, wg1, gi[:, :2 * _PH]))
        r1 = g1[:, :_PH]
        u1 = g1[:, _PH:]
        qmats = _diffuse(s1m, s2m, r1 * s1f)
        cand1 = jnp.tanh(_proj(qmats, wc1, gi[:, 2 * _PH:]))
        s1n = u1 * s1f + (1.0 - u1) * cand1

        g0 = jax.nn.sigmoid(_proj(dm0, wg0, g0c_ref[t + 1][:, :2 * _PH]))
        r0 = g0[:, :_PH]
        u0 = g0[:, _PH:]
        cmats = _diffuse(s1m, s2m, r0 * o0f)
        cand0 = jnp.tanh(_proj(cmats, wc0, g0c_ref[t + 1][:, 2 * _PH:]))
        o0n = u0 * o0f + (1.0 - u0) * cand0

        return (jnp.concatenate([o0n, s1n], axis=1), o0f)

    z = jnp.zeros((_N, _PH), jnp.float32)
    os0 = jnp.concatenate([o00, z], axis=1)
    os_fin, s0_fin = jax.lax.fori_loop(0, _T, step, (os0, z))
    for j in range(_P):
        out_ref[0, j] = s0_fin[:, j * _H:(j + 1) * _H]
        out_ref[1, j] = os_fin[:, _PH + j * _H:_PH + (j + 1) * _H]


def _bd_gate(w):
    r, u = w[:, :_H], w[:, _H:]
    z = jnp.zeros_like(r)
    return jnp.concatenate(
        [jnp.concatenate([r, z, u, z], axis=1),
         jnp.concatenate([z, r, z, u], axis=1)], axis=0)


def _bd_cand(w):
    z = jnp.zeros_like(w)
    return jnp.concatenate(
        [jnp.concatenate([w, z], axis=1),
         jnp.concatenate([z, w], axis=1)], axis=0)


def _bd_fused(wr, wh):
    r, u = wr[:, :_H], wr[:, _H:]
    z = jnp.zeros_like(r)
    return jnp.concatenate(
        [jnp.concatenate([r, z, u, z, wh, z], axis=1),
         jnp.concatenate([z, r, z, u, z, wh], axis=1)], axis=0)


def kernel(inputs, supports, W_ru_0, b_ru_0, W_h_0, b_h_0,
           W_ru_1, b_ru_1, W_h_1, b_h_1):
    x_g = inputs.transpose(1, 2, 0, 3)
    x_g = x_g.reshape(_B // _P, _P, _N, _T, _I)
    x_g = x_g.transpose(0, 2, 3, 1, 4).reshape(_B // _P, _N, _T * _P * _I)

    wg0 = jnp.concatenate(
        [_bd_gate(W_ru_0[k * _C0 + _I:(k + 1) * _C0])
         for k in range(_NUM_MAT)], axis=0)
    wc0 = jnp.concatenate(
        [_bd_cand(W_h_0[k * _C0 + _I:(k + 1) * _C0])
         for k in range(_NUM_MAT)], axis=0)
    wg1 = jnp.concatenate(
        [_bd_gate(W_ru_1[k * _C1 + _H:(k + 1) * _C1])
         for k in range(_NUM_MAT)], axis=0)
    wc1 = jnp.concatenate(
        [_bd_cand(W_h_1[k * _C1 + _H:(k + 1) * _C1])
         for k in range(_NUM_MAT)], axis=0)
    w1i = jnp.concatenate(
        [_bd_fused(W_ru_1[k * _C1:k * _C1 + _H],
                   W_h_1[k * _C1:k * _C1 + _H])
         for k in range(_NUM_MAT)], axis=0)

    blocks = []
    for k in range(_NUM_MAT):
        r = W_ru_0[k * _C0:k * _C0 + _I, :_H]
        u = W_ru_0[k * _C0:k * _C0 + _I, _H:]
        c = W_h_0[k * _C0:k * _C0 + _I]
        z = jnp.zeros_like(r)
        blocks.append(jnp.concatenate(
            [jnp.concatenate([r, z, u, z, c, z], axis=1),
             jnp.concatenate([z, r, z, u, z, c], axis=1)], axis=0))
    w0i = jnp.concatenate(blocks, axis=0)

    b0c = jnp.concatenate([b_ru_0[:_H], b_ru_0[:_H], b_ru_0[_H:],
                           b_ru_0[_H:], b_h_0, b_h_0]).reshape(1, 6 * _H)
    b1c = jnp.concatenate([b_ru_1[:_H], b_ru_1[:_H], b_ru_1[_H:],
                           b_ru_1[_H:], b_h_1, b_h_1]).reshape(1, 6 * _H)

    out = pl.pallas_call(
        _body,
        grid=(_B // _P,),
        in_specs=[
            pl.BlockSpec((1, _N, _T * _P * _I), lambda p: (p, 0, 0)),
            pl.BlockSpec((_S, _N, _N), lambda p: (0, 0, 0)),
            pl.BlockSpec((_NUM_MAT * _P * _I, 6 * _H), lambda p: (0, 0)),
            pl.BlockSpec((_NUM_MAT * 2 * _H, 4 * _H), lambda p: (0, 0)),
            pl.BlockSpec((_NUM_MAT * 2 * _H, 2 * _H), lambda p: (0, 0)),
            pl.BlockSpec((_NUM_MAT * 2 * _H, 6 * _H), lambda p: (0, 0)),
            pl.BlockSpec((_NUM_MAT * 2 * _H, 4 * _H), lambda p: (0, 0)),
            pl.BlockSpec((_NUM_MAT * 2 * _H, 2 * _H), lambda p: (0, 0)),
            pl.BlockSpec((1, 6 * _H), lambda p: (0, 0)),
            pl.BlockSpec((1, 6 * _H), lambda p: (0, 0)),
        ],
        out_specs=pl.BlockSpec((_L, _P, _N, _H), lambda p: (0, p, 0, 0)),
        out_shape=jax.ShapeDtypeStruct((_L, _B, _N, _H), jnp.float32),
        scratch_shapes=[pltpu.VMEM((_T + 1, _N, 6 * _H), jnp.float32)],
        compiler_params=pltpu.CompilerParams(
            dimension_semantics=("parallel",)),
    )(x_g, supports.astype(jnp.bfloat16),
      w0i.astype(jnp.bfloat16), wg0.astype(jnp.bfloat16),
      wc0.astype(jnp.bfloat16), w1i.astype(jnp.bfloat16),
      wg1.astype(jnp.bfloat16), wc1.astype(jnp.bfloat16), b0c, b1c)
    return out


# speedup vs baseline: 2.8840x; 1.2550x over previous
import jax
import jax.numpy as jnp
from jax.experimental import pallas as pl
from jax.experimental.pallas import tpu as pltpu

_T, _B, _N, _I = 12, 16, 512, 2
_H = 64
_L = 2
_S = 2
_K = 3
_NUM_MAT = 1 + _S * (_K - 1)
_P = 2
_C0 = _I + _H
_C1 = _H + _H
_PH = _P * _H


def _diffuse(s1, s2, x):
    xb = x.astype(jnp.bfloat16)
    t1a = jnp.dot(s1, xb, preferred_element_type=jnp.float32)
    t1ab = t1a.astype(jnp.bfloat16)
    t2ab = (2.0 * jnp.dot(s1, t1ab, preferred_element_type=jnp.float32)
            - x).astype(jnp.bfloat16)
    t1b = jnp.dot(s2, xb, preferred_element_type=jnp.float32)
    t1bb = t1b.astype(jnp.bfloat16)
    t2bb = (2.0 * jnp.dot(s2, t1bb, preferred_element_type=jnp.float32)
            - x).astype(jnp.bfloat16)
    return [xb, t1ab, t2ab, t1bb, t2bb]


def _proj(mats, w, acc):
    for k, m in enumerate(mats):
        acc = acc + jnp.dot(m, w[k * _PH:(k + 1) * _PH],
                            preferred_element_type=jnp.float32)
    return acc


def _body(x_ref, sup_ref, w0i_ref, wg0_ref, wc0_ref,
          w1i_ref, wg1_ref, wc1_ref, b0_ref, b1_ref,
          out_ref, g0c_ref):
    s1m = sup_ref[0]
    s2m = sup_ref[1]
    w0i = w0i_ref[:, :]
    wg0 = wg0_ref[:, :]
    wc0 = wc0_ref[:, :]
    w1i = w1i_ref[:, :]
    wg1 = wg1_ref[:, :]
    wc1 = wc1_ref[:, :]
    b0c = b0_ref[:, :]
    b1c = b1_ref[:, :]

    imats = _diffuse(s1m, s2m, x_ref[0])
    for t in range(_T):
        cols = jnp.concatenate(
            [m[:, _P * _I * t:_P * _I * (t + 1)] for m in imats], axis=1)
        g0c_ref[t] = jnp.dot(cols, w0i,
                             preferred_element_type=jnp.float32) + b0c
    g0c_ref[_T] = jnp.zeros((_N, 6 * _H), jnp.float32)

    g00 = g0c_ref[0]
    u00 = jax.nn.sigmoid(g00[:, _PH:2 * _PH])
    o00 = (1.0 - u00) * jnp.tanh(g00[:, 4 * _H:])

    def step(t, carry):
        os, _ = carry
        o0f = os[:, :_PH]
        s1f = os[:, _PH:]

        dmats = _diffuse(s1m, s2m, os)
        dm0 = [m[:, :_PH] for m in dmats]
        dm1 = [m[:, _PH:] for m in dmats]

        gi = _proj(dm0, w1i, b1c)
        g1 = jax.nn.sigmoid(_proj(dm1, wg1, gi[:, :2 * _PH]))
        r1 = g1[:, :_PH]
        u1 = g1[:, _PH:]
        g0 = jax.nn.sigmoid(_proj(dm0, wg0, g0c_ref[t + 1][:, :2 * _PH]))
        r0 = g0[:, :_PH]
        u0 = g0[:, _PH:]

        rc = jnp.concatenate([r0 * o0f, r1 * s1f], axis=1)
        rcmats = _diffuse(s1m, s2m, rc)
        cm0 = [m[:, :_PH] for m in rcmats]
        cm1 = [m[:, _PH:] for m in rcmats]
        cand0 = jnp.tanh(_proj(cm0, wc0, g0c_ref[t + 1][:, 2 * _PH:]))
        cand1 = jnp.tanh(_proj(cm1, wc1, gi[:, 2 * _PH:]))
        s1n = u1 * s1f + (1.0 - u1) * cand1
        o0n = u0 * o0f + (1.0 - u0) * cand0

        return (jnp.concatenate([o0n, s1n], axis=1), o0f)

    z = jnp.zeros((_N, _PH), jnp.float32)
    os0 = jnp.concatenate([o00, z], axis=1)
    os_fin, s0_fin = jax.lax.fori_loop(0, _T, step, (os0, z))
    for j in range(_P):
        out_ref[0, j] = s0_fin[:, j * _H:(j + 1) * _H]
        out_ref[1, j] = os_fin[:, _PH + j * _H:_PH + (j + 1) * _H]


def _bd_gate(w):
    r, u = w[:, :_H], w[:, _H:]
    z = jnp.zeros_like(r)
    return jnp.concatenate(
        [jnp.concatenate([r, z, u, z], axis=1),
         jnp.concatenate([z, r, z, u], axis=1)], axis=0)


def _bd_cand(w):
    z = jnp.zeros_like(w)
    return jnp.concatenate(
        [jnp.concatenate([w, z], axis=1),
         jnp.concatenate([z, w], axis=1)], axis=0)


def _bd_fused(wr, wh):
    r, u = wr[:, :_H], wr[:, _H:]
    z = jnp.zeros_like(r)
    return jnp.concatenate(
        [jnp.concatenate([r, z, u, z, wh, z], axis=1),
         jnp.concatenate([z, r, z, u, z, wh], axis=1)], axis=0)


def kernel(inputs, supports, W_ru_0, b_ru_0, W_h_0, b_h_0,
           W_ru_1, b_ru_1, W_h_1, b_h_1):
    x_g = inputs.transpose(1, 2, 0, 3)
    x_g = x_g.reshape(_B // _P, _P, _N, _T, _I)
    x_g = x_g.transpose(0, 2, 3, 1, 4).reshape(_B // _P, _N, _T * _P * _I)

    wg0 = jnp.concatenate(
        [_bd_gate(W_ru_0[k * _C0 + _I:(k + 1) * _C0])
         for k in range(_NUM_MAT)], axis=0)
    wc0 = jnp.concatenate(
        [_bd_cand(W_h_0[k * _C0 + _I:(k + 1) * _C0])
         for k in range(_NUM_MAT)], axis=0)
    wg1 = jnp.concatenate(
        [_bd_gate(W_ru_1[k * _C1 + _H:(k + 1) * _C1])
         for k in range(_NUM_MAT)], axis=0)
    wc1 = jnp.concatenate(
        [_bd_cand(W_h_1[k * _C1 + _H:(k + 1) * _C1])
         for k in range(_NUM_MAT)], axis=0)
    w1i = jnp.concatenate(
        [_bd_fused(W_ru_1[k * _C1:k * _C1 + _H],
                   W_h_1[k * _C1:k * _C1 + _H])
         for k in range(_NUM_MAT)], axis=0)

    blocks = []
    for k in range(_NUM_MAT):
        r = W_ru_0[k * _C0:k * _C0 + _I, :_H]
        u = W_ru_0[k * _C0:k * _C0 + _I, _H:]
        c = W_h_0[k * _C0:k * _C0 + _I]
        z = jnp.zeros_like(r)
        blocks.append(jnp.concatenate(
            [jnp.concatenate([r, z, u, z, c, z], axis=1),
             jnp.concatenate([z, r, z, u, z, c], axis=1)], axis=0))
    w0i = jnp.concatenate(blocks, axis=0)

    b0c = jnp.concatenate([b_ru_0[:_H], b_ru_0[:_H], b_ru_0[_H:],
                           b_ru_0[_H:], b_h_0, b_h_0]).reshape(1, 6 * _H)
    b1c = jnp.concatenate([b_ru_1[:_H], b_ru_1[:_H], b_ru_1[_H:],
                           b_ru_1[_H:], b_h_1, b_h_1]).reshape(1, 6 * _H)

    out = pl.pallas_call(
        _body,
        grid=(_B // _P,),
        in_specs=[
            pl.BlockSpec((1, _N, _T * _P * _I), lambda p: (p, 0, 0)),
            pl.BlockSpec((_S, _N, _N), lambda p: (0, 0, 0)),
            pl.BlockSpec((_NUM_MAT * _P * _I, 6 * _H), lambda p: (0, 0)),
            pl.BlockSpec((_NUM_MAT * 2 * _H, 4 * _H), lambda p: (0, 0)),
            pl.BlockSpec((_NUM_MAT * 2 * _H, 2 * _H), lambda p: (0, 0)),
            pl.BlockSpec((_NUM_MAT * 2 * _H, 6 * _H), lambda p: (0, 0)),
            pl.BlockSpec((_NUM_MAT * 2 * _H, 4 * _H), lambda p: (0, 0)),
            pl.BlockSpec((_NUM_MAT * 2 * _H, 2 * _H), lambda p: (0, 0)),
            pl.BlockSpec((1, 6 * _H), lambda p: (0, 0)),
            pl.BlockSpec((1, 6 * _H), lambda p: (0, 0)),
        ],
        out_specs=pl.BlockSpec((_L, _P, _N, _H), lambda p: (0, p, 0, 0)),
        out_shape=jax.ShapeDtypeStruct((_L, _B, _N, _H), jnp.float32),
        scratch_shapes=[pltpu.VMEM((_T + 1, _N, 6 * _H), jnp.float32)],
        compiler_params=pltpu.CompilerParams(
            dimension_semantics=("parallel",)),
    )(x_g, supports.astype(jnp.bfloat16),
      w0i.astype(jnp.bfloat16), wg0.astype(jnp.bfloat16),
      wc0.astype(jnp.bfloat16), w1i.astype(jnp.bfloat16),
      wg1.astype(jnp.bfloat16), wc1.astype(jnp.bfloat16), b0c, b1c)
    return out


# speedup vs baseline: 3.2864x; 1.1395x over previous
import jax
import jax.numpy as jnp
from jax.experimental import pallas as pl
from jax.experimental.pallas import tpu as pltpu

_T, _B, _N, _I = 12, 16, 512, 2
_H = 64
_L = 2
_S = 2
_K = 3
_NUM_MAT = 1 + _S * (_K - 1)
_P = 4
_Q = 2
_C0 = _I + _H
_C1 = _H + _H
_PH = 2 * _H
_G = 6 * _H


def _diffuse(s1, s2, x):
    xb = x.astype(jnp.bfloat16)
    t1a = jnp.dot(s1, xb, preferred_element_type=jnp.float32)
    t1ab = t1a.astype(jnp.bfloat16)
    t2ab = (2.0 * jnp.dot(s1, t1ab, preferred_element_type=jnp.float32)
            - x).astype(jnp.bfloat16)
    t1b = jnp.dot(s2, xb, preferred_element_type=jnp.float32)
    t1bb = t1b.astype(jnp.bfloat16)
    t2bb = (2.0 * jnp.dot(s2, t1bb, preferred_element_type=jnp.float32)
            - x).astype(jnp.bfloat16)
    return [xb, t1ab, t2ab, t1bb, t2bb]


def _proj(mats, w, acc):
    for k, m in enumerate(mats):
        acc = acc + jnp.dot(m, w[k * _PH:(k + 1) * _PH],
                            preferred_element_type=jnp.float32)
    return acc


def _body(x_ref, sup_ref, w0i_ref, wg0_ref, wc0_ref,
          w1i_ref, wg1_ref, wc1_ref, b0_ref, b1_ref,
          out_ref, g0c_ref):
    s1m = sup_ref[0]
    s2m = sup_ref[1]
    w0i = w0i_ref[:, :]
    wg0 = wg0_ref[:, :]
    wc0 = wc0_ref[:, :]
    w1i = w1i_ref[:, :]
    wg1 = wg1_ref[:, :]
    wc1 = wc1_ref[:, :]
    b0c = b0_ref[:, :]
    b1c = b1_ref[:, :]

    imats = _diffuse(s1m, s2m, x_ref[0])
    for t in range(_T):
        cols = jnp.concatenate(
            [m[:, _P * _I * t:_P * _I * (t + 1)] for m in imats], axis=1)
        g0c_ref[t] = jnp.dot(cols, w0i,
                             preferred_element_type=jnp.float32) + b0c
    g0c_ref[_T] = jnp.zeros((_N, _Q * _G), jnp.float32)

    g00 = g0c_ref[0]
    o00 = jnp.concatenate(
        [(1.0 - jax.nn.sigmoid(g00[:, q * _G + _PH:q * _G + 2 * _PH]))
         * jnp.tanh(g00[:, q * _G + 2 * _PH:(q + 1) * _G])
         for q in range(_Q)], axis=1)

    def step(t, carry):
        os, _ = carry
        qph = _Q * _PH

        dmats = _diffuse(s1m, s2m, os)
        g0n = g0c_ref[t + 1]

        rcs, u1s, u0s, gis = [], [], [], []
        for q in range(_Q):
            dm0 = [m[:, q * _PH:(q + 1) * _PH] for m in dmats]
            dm1 = [m[:, qph + q * _PH:qph + (q + 1) * _PH] for m in dmats]
            gi = _proj(dm0, w1i, b1c)
            g1 = jax.nn.sigmoid(_proj(dm1, wg1, gi[:, :2 * _PH]))
            g0 = jax.nn.sigmoid(
                _proj(dm0, wg0, g0n[:, q * _G:q * _G + 2 * _PH]))
            rcs.append((g0[:, :_PH] * os[:, q * _PH:(q + 1) * _PH],
                        g1[:, :_PH] * os[:, qph + q * _PH:
                                         qph + (q + 1) * _PH]))
            u1s.append(g1[:, _PH:])
            u0s.append(g0[:, _PH:])
            gis.append(gi)

        rc = jnp.concatenate([rcs[0][0], rcs[1][0],
                              rcs[0][1], rcs[1][1]], axis=1)
        rcmats = _diffuse(s1m, s2m, rc)

        outs = []
        for q, gi in enumerate(gis):
            cm0 = [m[:, q * _PH:(q + 1) * _PH] for m in rcmats]
            cm1 = [m[:, qph + q * _PH:qph + (q + 1) * _PH]
                   for m in rcmats]
            cand0 = jnp.tanh(
                _proj(cm0, wc0, g0n[:, q * _G + 2 * _PH:(q + 1) * _G]))
            cand1 = jnp.tanh(_proj(cm1, wc1, gi[:, 2 * _PH:]))
            o0f = os[:, q * _PH:(q + 1) * _PH]
            s1f = os[:, qph + q * _PH:qph + (q + 1) * _PH]
            outs.append((u0s[q] * o0f + (1.0 - u0s[q]) * cand0,
                         u1s[q] * s1f + (1.0 - u1s[q]) * cand1))

        os_n = jnp.concatenate([outs[0][0], outs[1][0],
                                outs[0][1], outs[1][1]], axis=1)
        return (os_n, os[:, :qph])

    z = jnp.zeros((_N, _Q * _PH), jnp.float32)
    os0 = jnp.concatenate([o00, z], axis=1)
    os_fin, s0_fin = jax.lax.fori_loop(0, _T, step, (os0, z))
    for j in range(_P):
        out_ref[0, j] = s0_fin[:, j * _H:(j + 1) * _H]
        out_ref[1, j] = os_fin[:, _Q * _PH + j * _H:
                               _Q * _PH + (j + 1) * _H]


def _bd_gate(w):
    r, u = w[:, :_H], w[:, _H:]
    z = jnp.zeros_like(r)
    return jnp.concatenate(
        [jnp.concatenate([r, z, u, z], axis=1),
         jnp.concatenate([z, r, z, u], axis=1)], axis=0)


def _bd_cand(w):
    z = jnp.zeros_like(w)
    return jnp.concatenate(
        [jnp.concatenate([w, z], axis=1),
         jnp.concatenate([z, w], axis=1)], axis=0)


def _bd_fused(wr, wh):
    r, u = wr[:, :_H], wr[:, _H:]
    z = jnp.zeros_like(r)
    return jnp.concatenate(
        [jnp.concatenate([r, z, u, z, wh, z], axis=1),
         jnp.concatenate([z, r, z, u, z, wh], axis=1)], axis=0)


def kernel(inputs, supports, W_ru_0, b_ru_0, W_h_0, b_h_0,
           W_ru_1, b_ru_1, W_h_1, b_h_1):
    x_g = inputs.transpose(1, 2, 0, 3)
    x_g = x_g.reshape(_B // _P, _P, _N, _T, _I)
    x_g = x_g.transpose(0, 2, 3, 1, 4).reshape(_B // _P, _N, _T * _P * _I)

    wg0 = jnp.concatenate(
        [_bd_gate(W_ru_0[k * _C0 + _I:(k + 1) * _C0])
         for k in range(_NUM_MAT)], axis=0)
    wc0 = jnp.concatenate(
        [_bd_cand(W_h_0[k * _C0 + _I:(k + 1) * _C0])
         for k in range(_NUM_MAT)], axis=0)
    wg1 = jnp.concatenate(
        [_bd_gate(W_ru_1[k * _C1 + _H:(k + 1) * _C1])
         for k in range(_NUM_MAT)], axis=0)
    wc1 = jnp.concatenate(
        [_bd_cand(W_h_1[k * _C1 + _H:(k + 1) * _C1])
         for k in range(_NUM_MAT)], axis=0)
    w1i = jnp.concatenate(
        [_bd_fused(W_ru_1[k * _C1:k * _C1 + _H],
                   W_h_1[k * _C1:k * _C1 + _H])
         for k in range(_NUM_MAT)], axis=0)

    blocks = []
    for k in range(_NUM_MAT):
        r = W_ru_0[k * _C0:k * _C0 + _I, :_H]
        u = W_ru_0[k * _C0:k * _C0 + _I, _H:]
        c = W_h_0[k * _C0:k * _C0 + _I]
        z = jnp.zeros_like(r)
        pair = jnp.concatenate(
            [jnp.concatenate([r, z, u, z, c, z], axis=1),
             jnp.concatenate([z, r, z, u, z, c], axis=1)], axis=0)
        zz = jnp.zeros_like(pair)
        blocks.append(jnp.concatenate(
            [jnp.concatenate([pair, zz], axis=1),
             jnp.concatenate([zz, pair], axis=1)], axis=0))
    w0i = jnp.concatenate(blocks, axis=0)

    b0q = jnp.concatenate([b_ru_0[:_H], b_ru_0[:_H], b_ru_0[_H:],
                           b_ru_0[_H:], b_h_0, b_h_0])
    b0c = jnp.concatenate([b0q, b0q]).reshape(1, _Q * 6 * _H)
    b1c = jnp.concatenate([b_ru_1[:_H], b_ru_1[:_H], b_ru_1[_H:],
                           b_ru_1[_H:], b_h_1, b_h_1]).reshape(1, 6 * _H)

    out = pl.pallas_call(
        _body,
        grid=(_B // _P,),
        in_specs=[
            pl.BlockSpec((1, _N, _T * _P * _I), lambda p: (p, 0, 0)),
            pl.BlockSpec((_S, _N, _N), lambda p: (0, 0, 0)),
            pl.BlockSpec((_NUM_MAT * _P * _I, _Q * 6 * _H),
                         lambda p: (0, 0)),
            pl.BlockSpec((_NUM_MAT * 2 * _H, 4 * _H), lambda p: (0, 0)),
            pl.BlockSpec((_NUM_MAT * 2 * _H, 2 * _H), lambda p: (0, 0)),
            pl.BlockSpec((_NUM_MAT * 2 * _H, 6 * _H), lambda p: (0, 0)),
            pl.BlockSpec((_NUM_MAT * 2 * _H, 4 * _H), lambda p: (0, 0)),
            pl.BlockSpec((_NUM_MAT * 2 * _H, 2 * _H), lambda p: (0, 0)),
            pl.BlockSpec((1, _Q * 6 * _H), lambda p: (0, 0)),
            pl.BlockSpec((1, 6 * _H), lambda p: (0, 0)),
        ],
        out_specs=pl.BlockSpec((_L, _P, _N, _H), lambda p: (0, p, 0, 0)),
        out_shape=jax.ShapeDtypeStruct((_L, _B, _N, _H), jnp.float32),
        scratch_shapes=[pltpu.VMEM((_T + 1, _N, _Q * 6 * _H), jnp.float32)],
        compiler_params=pltpu.CompilerParams(
            dimension_semantics=("parallel",)),
    )(x_g, supports.astype(jnp.bfloat16),
      w0i.astype(jnp.bfloat16), wg0.astype(jnp.bfloat16),
      wc0.astype(jnp.bfloat16), w1i.astype(jnp.bfloat16),
      wg1.astype(jnp.bfloat16), wc1.astype(jnp.bfloat16), b0c, b1c)
    return out


# speedup vs baseline: 3.7495x; 1.1409x over previous
import jax
import jax.numpy as jnp
from jax.experimental import pallas as pl
from jax.experimental.pallas import tpu as pltpu

_T, _B, _N, _I = 12, 16, 512, 2
_H = 64
_L = 2
_S = 2
_K = 3
_NUM_MAT = 1 + _S * (_K - 1)
_P = 8
_Q = 4
_C0 = _I + _H
_C1 = _H + _H
_PH = 2 * _H
_G = 6 * _H


def _diffuse(s1, s2, x):
    xb = x.astype(jnp.bfloat16)
    t1a = jnp.dot(s1, xb, preferred_element_type=jnp.float32)
    t1ab = t1a.astype(jnp.bfloat16)
    t2ab = (2.0 * jnp.dot(s1, t1ab, preferred_element_type=jnp.float32)
            - x).astype(jnp.bfloat16)
    t1b = jnp.dot(s2, xb, preferred_element_type=jnp.float32)
    t1bb = t1b.astype(jnp.bfloat16)
    t2bb = (2.0 * jnp.dot(s2, t1bb, preferred_element_type=jnp.float32)
            - x).astype(jnp.bfloat16)
    return [xb, t1ab, t2ab, t1bb, t2bb]


def _proj(mats, w, acc):
    for k, m in enumerate(mats):
        acc = acc + jnp.dot(m, w[k * _PH:(k + 1) * _PH],
                            preferred_element_type=jnp.float32)
    return acc


def _body(x_ref, sup_ref, w0i_ref, wg0_ref, wc0_ref,
          w1i_ref, wg1_ref, wc1_ref, b0_ref, b1_ref,
          out_ref, g0c_ref):
    s1m = sup_ref[0]
    s2m = sup_ref[1]
    w0i = w0i_ref[:, :]
    wg0 = wg0_ref[:, :]
    wc0 = wc0_ref[:, :]
    w1i = w1i_ref[:, :]
    wg1 = wg1_ref[:, :]
    wc1 = wc1_ref[:, :]
    b0c = b0_ref[:, :]
    b1c = b1_ref[:, :]

    imats = _diffuse(s1m, s2m, x_ref[0])
    for t in range(_T):
        cols = jnp.concatenate(
            [m[:, _P * _I * t:_P * _I * (t + 1)] for m in imats], axis=1)
        g0c_ref[t] = (jnp.dot(cols, w0i, preferred_element_type=jnp.float32)
                      + b0c).astype(jnp.bfloat16)
    g0c_ref[_T] = jnp.zeros((_N, _Q * _G), jnp.bfloat16)

    g00 = g0c_ref[0].astype(jnp.float32)
    o00 = jnp.concatenate(
        [(1.0 - jax.nn.sigmoid(g00[:, q * _G + _PH:q * _G + 2 * _PH]))
         * jnp.tanh(g00[:, q * _G + 2 * _PH:(q + 1) * _G])
         for q in range(_Q)], axis=1)

    def step(t, carry):
        os, _ = carry
        qph = _Q * _PH

        dmats = _diffuse(s1m, s2m, os)
        g0n = g0c_ref[t + 1].astype(jnp.float32)

        rcs, u1s, u0s, gis = [], [], [], []
        for q in range(_Q):
            dm0 = [m[:, q * _PH:(q + 1) * _PH] for m in dmats]
            dm1 = [m[:, qph + q * _PH:qph + (q + 1) * _PH] for m in dmats]
            gi = _proj(dm0, w1i, b1c)
            g1 = jax.nn.sigmoid(_proj(dm1, wg1, gi[:, :2 * _PH]))
            g0 = jax.nn.sigmoid(
                _proj(dm0, wg0, g0n[:, q * _G:q * _G + 2 * _PH]))
            rcs.append((g0[:, :_PH] * os[:, q * _PH:(q + 1) * _PH],
                        g1[:, :_PH] * os[:, qph + q * _PH:
                                         qph + (q + 1) * _PH]))
            u1s.append(g1[:, _PH:])
            u0s.append(g0[:, _PH:])
            gis.append(gi)

        rc = jnp.concatenate([rcs[q][0] for q in range(_Q)]
                             + [rcs[q][1] for q in range(_Q)], axis=1)
        rcmats = _diffuse(s1m, s2m, rc)

        outs = []
        for q, gi in enumerate(gis):
            cm0 = [m[:, q * _PH:(q + 1) * _PH] for m in rcmats]
            cm1 = [m[:, qph + q * _PH:qph + (q + 1) * _PH]
                   for m in rcmats]
            cand0 = jnp.tanh(
                _proj(cm0, wc0, g0n[:, q * _G + 2 * _PH:(q + 1) * _G]))
            cand1 = jnp.tanh(_proj(cm1, wc1, gi[:, 2 * _PH:]))
            o0f = os[:, q * _PH:(q + 1) * _PH]
            s1f = os[:, qph + q * _PH:qph + (q + 1) * _PH]
            outs.append((u0s[q] * o0f + (1.0 - u0s[q]) * cand0,
                         u1s[q] * s1f + (1.0 - u1s[q]) * cand1))

        os_n = jnp.concatenate([outs[q][0] for q in range(_Q)]
                               + [outs[q][1] for q in range(_Q)], axis=1)
        return (os_n, os[:, :qph])

    z = jnp.zeros((_N, _Q * _PH), jnp.float32)
    os0 = jnp.concatenate([o00, z], axis=1)
    os_fin, s0_fin = jax.lax.fori_loop(0, _T, step, (os0, z))
    for j in range(_P):
        out_ref[0, j] = s0_fin[:, j * _H:(j + 1) * _H]
        out_ref[1, j] = os_fin[:, _Q * _PH + j * _H:
                               _Q * _PH + (j + 1) * _H]


def _bd_gate(w):
    r, u = w[:, :_H], w[:, _H:]
    z = jnp.zeros_like(r)
    return jnp.concatenate(
        [jnp.concatenate([r, z, u, z], axis=1),
         jnp.concatenate([z, r, z, u], axis=1)], axis=0)


def _bd_cand(w):
    z = jnp.zeros_like(w)
    return jnp.concatenate(
        [jnp.concatenate([w, z], axis=1),
         jnp.concatenate([z, w], axis=1)], axis=0)


def _bd_fused(wr, wh):
    r, u = wr[:, :_H], wr[:, _H:]
    z = jnp.zeros_like(r)
    return jnp.concatenate(
        [jnp.concatenate([r, z, u, z, wh, z], axis=1),
         jnp.concatenate([z, r, z, u, z, wh], axis=1)], axis=0)


def kernel(inputs, supports, W_ru_0, b_ru_0, W_h_0, b_h_0,
           W_ru_1, b_ru_1, W_h_1, b_h_1):
    x_g = inputs.transpose(1, 2, 0, 3)
    x_g = x_g.reshape(_B // _P, _P, _N, _T, _I)
    x_g = x_g.transpose(0, 2, 3, 1, 4).reshape(_B // _P, _N, _T * _P * _I)

    wg0 = jnp.concatenate(
        [_bd_gate(W_ru_0[k * _C0 + _I:(k + 1) * _C0])
         for k in range(_NUM_MAT)], axis=0)
    wc0 = jnp.concatenate(
        [_bd_cand(W_h_0[k * _C0 + _I:(k + 1) * _C0])
         for k in range(_NUM_MAT)], axis=0)
    wg1 = jnp.concatenate(
        [_bd_gate(W_ru_1[k * _C1 + _H:(k + 1) * _C1])
         for k in range(_NUM_MAT)], axis=0)
    wc1 = jnp.concatenate(
        [_bd_cand(W_h_1[k * _C1 + _H:(k + 1) * _C1])
         for k in range(_NUM_MAT)], axis=0)
    w1i = jnp.concatenate(
        [_bd_fused(W_ru_1[k * _C1:k * _C1 + _H],
                   W_h_1[k * _C1:k * _C1 + _H])
         for k in range(_NUM_MAT)], axis=0)

    blocks = []
    for k in range(_NUM_MAT):
        r = W_ru_0[k * _C0:k * _C0 + _I, :_H]
        u = W_ru_0[k * _C0:k * _C0 + _I, _H:]
        c = W_h_0[k * _C0:k * _C0 + _I]
        z = jnp.zeros_like(r)
        pair = jnp.concatenate(
            [jnp.concatenate([r, z, u, z, c, z], axis=1),
             jnp.concatenate([z, r, z, u, z, c], axis=1)], axis=0)
        zz = jnp.zeros_like(pair)
        blocks.append(jnp.concatenate(
            [jnp.concatenate([pair if i == q else zz for i in range(_Q)],
                             axis=1) for q in range(_Q)], axis=0))
    w0i = jnp.concatenate(blocks, axis=0)

    b0q = jnp.concatenate([b_ru_0[:_H], b_ru_0[:_H], b_ru_0[_H:],
                           b_ru_0[_H:], b_h_0, b_h_0])
    b0c = jnp.concatenate([b0q] * _Q).reshape(1, _Q * 6 * _H)
    b1c = jnp.concatenate([b_ru_1[:_H], b_ru_1[:_H], b_ru_1[_H:],
                           b_ru_1[_H:], b_h_1, b_h_1]).reshape(1, 6 * _H)

    out = pl.pallas_call(
        _body,
        grid=(_B // _P,),
        in_specs=[
            pl.BlockSpec((1, _N, _T * _P * _I), lambda p: (p, 0, 0)),
            pl.BlockSpec((_S, _N, _N), lambda p: (0, 0, 0)),
            pl.BlockSpec((_NUM_MAT * _P * _I, _Q * 6 * _H),
                         lambda p: (0, 0)),
            pl.BlockSpec((_NUM_MAT * 2 * _H, 4 * _H), lambda p: (0, 0)),
            pl.BlockSpec((_NUM_MAT * 2 * _H, 2 * _H), lambda p: (0, 0)),
            pl.BlockSpec((_NUM_MAT * 2 * _H, 6 * _H), lambda p: (0, 0)),
            pl.BlockSpec((_NUM_MAT * 2 * _H, 4 * _H), lambda p: (0, 0)),
            pl.BlockSpec((_NUM_MAT * 2 * _H, 2 * _H), lambda p: (0, 0)),
            pl.BlockSpec((1, _Q * 6 * _H), lambda p: (0, 0)),
            pl.BlockSpec((1, 6 * _H), lambda p: (0, 0)),
        ],
        out_specs=pl.BlockSpec((_L, _P, _N, _H), lambda p: (0, p, 0, 0)),
        out_shape=jax.ShapeDtypeStruct((_L, _B, _N, _H), jnp.float32),
        scratch_shapes=[pltpu.VMEM((_T + 1, _N, _Q * 6 * _H),
                                   jnp.bfloat16)],
        compiler_params=pltpu.CompilerParams(
            dimension_semantics=("parallel",)),
    )(x_g, supports.astype(jnp.bfloat16),
      w0i.astype(jnp.bfloat16), wg0.astype(jnp.bfloat16),
      wc0.astype(jnp.bfloat16), w1i.astype(jnp.bfloat16),
      wg1.astype(jnp.bfloat16), wc1.astype(jnp.bfloat16), b0c, b1c)
    return out
